# baseline (speedup 1.0000x reference)
import jax
import jax.numpy as jnp
from jax import lax
from jax.experimental import pallas as pl
from jax.experimental.pallas import tpu as pltpu
from jax.experimental.pallas import tpu_sc as plsc

B = 16384
L = 200
D = 32
NC = 2
NS = 16
NW = NC * NS
PB = B // NW
G = 4
RG = G * L
UB = 16
HU = PB // 2
HB = HU // UB


def _fism_body(his_flat_hbm, pre_hbm, users_hbm, scale_hbm,
               his_tab_hbm, item_tab_hbm, ubias_hbm, ibias_hbm,
               out_hbm,
               idx_half, rows0, rows1, pre_v, users_v, item_rows, scale_v,
               ubias_v, ibias_v, out_v, prod_buf,
               semr0, semr1, sem2):
    wid = lax.axis_index("s") * NC + lax.axis_index("c")
    base = wid * PB
    rows = (rows0, rows1)
    sems = (semr0, semr1)

    pltpu.sync_copy(pre_hbm.at[pl.ds(base, PB)], pre_v)
    pltpu.sync_copy(users_hbm.at[pl.ds(base, PB)], users_v)
    pltpu.sync_copy(scale_hbm.at[pl.ds(base, PB)], scale_v)
    cp_items = pltpu.async_copy(item_tab_hbm.at[pre_v], item_rows, sem2)
    cp_ub = pltpu.async_copy(ubias_hbm.at[users_v], ubias_v, sem2)
    cp_ib = pltpu.async_copy(ibias_hbm.at[pre_v], ibias_v, sem2)

    lane16 = lax.iota(jnp.int32, 16) * 16

    def fire(sg, b):
        pltpu.async_copy(
            his_tab_hbm.at[idx_half.at[pl.ds(sg * RG, RG)]], rows[b], sems[b])

    def wait(b):
        pltpu.make_async_copy(
            his_tab_hbm.at[idx_half.at[pl.ds(0, RG)]], rows[b], sems[b]).wait()

    cp_items.wait()
    cp_ub.wait()
    cp_ib.wait()

    @pl.loop(0, 2)
    def _(h):
        ho = h * HU
        pltpu.sync_copy(his_flat_hbm.at[pl.ds((base + ho) * L, HU * L)],
                        idx_half)
        fire(0, 0)

        @pl.loop(0, HB)
        def _(blk):
            for s in range(4):
                b = s % 2
                sg = blk * 4 + s
                if s < 3:
                    fire(sg + 1, 1 - b)
                else:
                    @pl.when(blk < HB - 1)
                    def _():
                        fire(sg + 1, 1 - b)
                wait(b)
                rv = rows[b]

                @pl.loop(0, G)
                def _(u):
                    init = (jnp.zeros((16,), jnp.float32),) * 8

                    @pl.loop(0, L, step=8, unroll=5, init_carry=init)
                    def pool(j, accs):
                        accs = list(accs)
                        for k in range(8):
                            r = u * L + j + k
                            c = k % 4
                            accs[c] = accs[c] + rv[r, pl.ds(0, 16)]
                            accs[4 + c] = accs[4 + c] + rv[r, pl.ds(16, 16)]
                        return tuple(accs)

                    lo = (pool[0] + pool[1]) + (pool[2] + pool[3])
                    hi = (pool[4] + pool[5]) + (pool[6] + pool[7])
                    uu = ho + blk * UB + s * G + u
                    prod = (lo * item_rows[uu, pl.ds(0, 16)]
                            + hi * item_rows[uu, pl.ds(16, 16)])
                    prod_buf[pl.ds((s * G + u) * 16, 16)] = prod

            rating = jnp.zeros((16,), jnp.float32)
            for d in range(16):
                rating = rating + plsc.load_gather(prod_buf, [lane16 + d])
            b0 = ho + blk * UB
            rating = (rating * scale_v[pl.ds(b0, UB)]
                      + ubias_v[pl.ds(b0, UB)] + ibias_v[pl.ds(b0, UB)])
            out_v[pl.ds(b0, UB)] = rating

    pltpu.sync_copy(out_v, out_hbm.at[pl.ds(base, PB)])


NI = 1000000
GS = 256
NGF = NI // GS
NTAIL = NI - NGF * GS
NLI = (NGF // NW + 1 + 1) // 2


def _linearize_body(a_hbm, b_hbm, oa_hbm, ob_hbm,
                    st0, st1, ob0, ob1, tb,
                    isem0, isem1, osem0, osem1):
    wid = lax.axis_index("s") * NC + lax.axis_index("c")
    lane = lax.iota(jnp.int32, 16)
    stage = (st0, st1)
    outb = (ob0, ob1)
    isems = (isem0, isem1)
    osems = (osem0, osem1)

    for in_hbm, out_hbm in ((a_hbm, oa_hbm), (b_hbm, ob_hbm)):
        def fire_in(g, b):
            for dt in range(4):
                pltpu.async_copy(
                    in_hbm.at[pl.ds(dt * 8, 8), pl.ds(g * GS, GS)],
                    stage[b].at[pl.ds(dt * 8, 8), pl.ds(0, GS)], isems[b])

        def wait_in(b):
            for dt in range(4):
                pltpu.make_async_copy(
                    in_hbm.at[pl.ds(0, 8), pl.ds(0, GS)],
                    stage[b].at[pl.ds(dt * 8, 8), pl.ds(0, GS)],
                    isems[b]).wait()

        def wait_out(b):
            pltpu.make_async_copy(
                outb[b], out_hbm.at[pl.ds(0, GS * D)], osems[b]).wait()

        fire_in(wid, 0)

        @pl.loop(0, NLI)
        def _(li):
            for p in range(2):
                g = wid + (li * 2 + p) * NW

                @pl.when(g + NW < NGF)
                def _():
                    fire_in(g + NW, 1 - p)

                @pl.when(g < NGF)
                def _():
                    wait_in(p)

                    @pl.when(g >= 2 * NW)
                    def _():
                        wait_out(p)

                    @plsc.parallel_loop(0, GS, unroll=8)
                    def _(r):
                        rr = jnp.full((16,), r, jnp.int32)
                        for dh in range(2):
                            v = plsc.load_gather(
                                stage[p], [lane + dh * 16, rr])
                            outb[p][pl.ds(r * D + dh * 16, 16)] = v

                    pltpu.async_copy(outb[p],
                                     out_hbm.at[pl.ds(g * (GS * D), GS * D)],
                                     osems[p])

        wait_out(0)
        wait_out(1)

        @pl.when(wid == 31)
        def _():
            for dt in range(4):
                pltpu.sync_copy(
                    in_hbm.at[pl.ds(dt * 8, 8), pl.ds(NGF * GS, NTAIL)],
                    tb.at[dt])

            @pl.loop(0, NTAIL)
            def _(r):
                rr = jnp.full((16,), r, jnp.int32)
                for dh in range(2):
                    v = plsc.load_gather(
                        tb, [lane // 8 + dh * 2, lane % 8, rr])
                    outb[0][pl.ds(r * D + dh * 16, 16)] = v
            pltpu.sync_copy(outb[0].at[pl.ds(0, NTAIL * 32)],
                            out_hbm.at[pl.ds(NGF * GS * 32, NTAIL * 32)])


def _linearize(his_t, item_t):
    mesh = plsc.VectorSubcoreMesh(core_axis_name="c", subcore_axis_name="s")
    fn = pl.kernel(
        _linearize_body,
        out_type=(jax.ShapeDtypeStruct((NI * D,), jnp.float32),
                  jax.ShapeDtypeStruct((NI * D,), jnp.float32)),
        mesh=mesh,
        compiler_params=pltpu.CompilerParams(
            needs_layout_passes=False, use_tc_tiling_on_sc=True),
        scratch_types=[
            pltpu.VMEM((D, GS + 1), jnp.float32),
            pltpu.VMEM((D, GS + 1), jnp.float32),
            pltpu.VMEM((GS * D,), jnp.float32),
            pltpu.VMEM((GS * D,), jnp.float32),
            pltpu.VMEM((4, 8, NTAIL), jnp.float32),
            pltpu.SemaphoreType.DMA,
            pltpu.SemaphoreType.DMA,
            pltpu.SemaphoreType.DMA,
            pltpu.SemaphoreType.DMA,
        ],
    )
    return fn(his_t, item_t)


def kernel(users, his_items, his_lens, pre_items, his_emb_table,
           item_emb_table, user_bias_table, item_bias_table):
    scale = jnp.power(his_lens, -0.5).astype(jnp.float32)
    his_flat = his_items.reshape(B * L).astype(jnp.int32)
    pre = pre_items.astype(jnp.int32)
    usr = users.astype(jnp.int32)
    ub = jnp.reshape(user_bias_table.T, (NI,))
    ib = jnp.reshape(item_bias_table.T, (NI,))
    his_lin, item_lin = _linearize(his_emb_table.T, item_emb_table.T)
    his_tab = his_lin.reshape(NI, D)
    item_tab = item_lin.reshape(NI, D)
    mesh = plsc.VectorSubcoreMesh(core_axis_name="c", subcore_axis_name="s")
    fn = pl.kernel(
        _fism_body,
        out_type=jax.ShapeDtypeStruct((B,), jnp.float32),
        mesh=mesh,
        compiler_params=pltpu.CompilerParams(
            needs_layout_passes=False, use_tc_tiling_on_sc=False),
        scratch_types=[
            pltpu.VMEM((HU * L,), jnp.int32),
            pltpu.VMEM((RG, D), jnp.float32),
            pltpu.VMEM((RG, D), jnp.float32),
            pltpu.VMEM((PB,), jnp.int32),
            pltpu.VMEM((PB,), jnp.int32),
            pltpu.VMEM((PB, D), jnp.float32),
            pltpu.VMEM((PB,), jnp.float32),
            pltpu.VMEM((PB,), jnp.float32),
            pltpu.VMEM((PB,), jnp.float32),
            pltpu.VMEM((PB,), jnp.float32),
            pltpu.VMEM((UB * 16,), jnp.float32),
            pltpu.SemaphoreType.DMA,
            pltpu.SemaphoreType.DMA,
            pltpu.SemaphoreType.DMA,
        ],
    )
    return fn(his_flat, pre, usr, scale, his_tab, item_tab, ub, ib)


# speedup vs baseline: 2.1905x; 2.1905x over previous
import jax
import jax.numpy as jnp
from jax import lax
from jax.experimental import pallas as pl
from jax.experimental.pallas import tpu as pltpu
from jax.experimental.pallas import tpu_sc as plsc

B = 16384
L = 200
D = 32
NC = 2
NS = 16
NW = NC * NS
PB = B // NW
G = 4
RG = G * L
UB = 16
HU = PB // 2
HB = HU // UB


def _fism_body(his_flat_hbm, pre_hbm, users_hbm, scale_hbm,
               his_tab_hbm, item_tab_hbm, ubias_hbm, ibias_hbm,
               out_hbm,
               idx_half, rows0, rows1, pre_v, users_v, item_rows, scale_v,
               ubias_v, ibias_v, out_v, prod_buf,
               semr0, semr1, sem2):
    wid = lax.axis_index("s") * NC + lax.axis_index("c")
    base = wid * PB
    rows = (rows0, rows1)
    sems = (semr0, semr1)

    pltpu.sync_copy(pre_hbm.at[pl.ds(base, PB)], pre_v)
    pltpu.sync_copy(users_hbm.at[pl.ds(base, PB)], users_v)
    pltpu.sync_copy(scale_hbm.at[pl.ds(base, PB)], scale_v)
    cp_items = pltpu.async_copy(item_tab_hbm.at[pre_v], item_rows, sem2)
    cp_ub = pltpu.async_copy(ubias_hbm.at[users_v], ubias_v, sem2)
    cp_ib = pltpu.async_copy(ibias_hbm.at[pre_v], ibias_v, sem2)

    lane16 = lax.iota(jnp.int32, 16) * 16

    def fire(sg, b):
        pltpu.async_copy(
            his_tab_hbm.at[idx_half.at[pl.ds(sg * RG, RG)]], rows[b], sems[b])

    def wait(b):
        pltpu.make_async_copy(
            his_tab_hbm.at[idx_half.at[pl.ds(0, RG)]], rows[b], sems[b]).wait()

    cp_items.wait()
    cp_ub.wait()
    cp_ib.wait()

    @pl.loop(0, 2)
    def _(h):
        ho = h * HU
        pltpu.sync_copy(his_flat_hbm.at[pl.ds((base + ho) * L, HU * L)],
                        idx_half)
        fire(0, 0)

        @pl.loop(0, HB)
        def _(blk):
            for s in range(4):
                b = s % 2
                sg = blk * 4 + s
                if s < 3:
                    fire(sg + 1, 1 - b)
                else:
                    @pl.when(blk < HB - 1)
                    def _():
                        fire(sg + 1, 1 - b)
                wait(b)
                rv = rows[b]

                @pl.loop(0, G)
                def _(u):
                    init = (jnp.zeros((16,), jnp.float32),) * 8

                    @pl.loop(0, L, step=8, unroll=5, init_carry=init)
                    def pool(j, accs):
                        accs = list(accs)
                        for k in range(8):
                            r = u * L + j + k
                            c = k % 4
                            accs[c] = accs[c] + rv[r, pl.ds(0, 16)]
                            accs[4 + c] = accs[4 + c] + rv[r, pl.ds(16, 16)]
                        return tuple(accs)

                    lo = (pool[0] + pool[1]) + (pool[2] + pool[3])
                    hi = (pool[4] + pool[5]) + (pool[6] + pool[7])
                    uu = ho + blk * UB + s * G + u
                    prod = (lo * item_rows[uu, pl.ds(0, 16)]
                            + hi * item_rows[uu, pl.ds(16, 16)])
                    prod_buf[pl.ds((s * G + u) * 16, 16)] = prod

            rating = jnp.zeros((16,), jnp.float32)
            for d in range(16):
                rating = rating + plsc.load_gather(prod_buf, [lane16 + d])
            b0 = ho + blk * UB
            rating = (rating * scale_v[pl.ds(b0, UB)]
                      + ubias_v[pl.ds(b0, UB)] + ibias_v[pl.ds(b0, UB)])
            out_v[pl.ds(b0, UB)] = rating

    pltpu.sync_copy(out_v, out_hbm.at[pl.ds(base, PB)])


NI = 1000000
GS = 256
NGF = NI // GS
NTAIL = NI - NGF * GS
NLI = (NGF // NW + 1 + 1) // 2


def _linearize_body(a_hbm, b_hbm, oa_hbm, ob_hbm,
                    st0, st1, ob0, ob1, tb,
                    isem0, isem1, osem0, osem1):
    wid = lax.axis_index("s") * NC + lax.axis_index("c")
    lane = lax.iota(jnp.int32, 16)
    lq4 = lane % 4
    sq4 = (lane // 4) * D + lane % 4
    stage = (st0, st1)
    outb = (ob0, ob1)
    isems = (isem0, isem1)
    osems = (osem0, osem1)

    for in_hbm, out_hbm in ((a_hbm, oa_hbm), (b_hbm, ob_hbm)):
        def fire_in(g, b):
            for dt in range(4):
                pltpu.async_copy(
                    in_hbm.at[pl.ds(dt * 8, 8), pl.ds(g * GS, GS)],
                    stage[b].at[pl.ds(dt * 8, 8), pl.ds(0, GS)], isems[b])

        def wait_in(b):
            for dt in range(4):
                pltpu.make_async_copy(
                    in_hbm.at[pl.ds(0, 8), pl.ds(0, GS)],
                    stage[b].at[pl.ds(dt * 8, 8), pl.ds(0, GS)],
                    isems[b]).wait()

        def wait_out(b):
            pltpu.make_async_copy(
                outb[b], out_hbm.at[pl.ds(0, GS * D)], osems[b]).wait()

        fire_in(wid, 0)

        @pl.loop(0, NLI)
        def _(li):
            for p in range(2):
                g = wid + (li * 2 + p) * NW

                @pl.when(g + NW < NGF)
                def _():
                    fire_in(g + NW, 1 - p)

                @pl.when(g < NGF)
                def _():
                    wait_in(p)

                    @pl.when(g >= 2 * NW)
                    def _():
                        wait_out(p)

                    @plsc.parallel_loop(0, GS // 4, unroll=4)
                    def _(rq):
                        r0 = rq * 4
                        for dq in range(8):
                            v = plsc.load_gather(
                                stage[p], [lq4 + dq * 4, lane // 4 + r0])
                            plsc.store_scatter(
                                outb[p],
                                [sq4 + (r0 * D + dq * 4)], v)

                    pltpu.async_copy(outb[p],
                                     out_hbm.at[pl.ds(g * (GS * D), GS * D)],
                                     osems[p])

        wait_out(0)
        wait_out(1)

        @pl.when(wid == 31)
        def _():
            for dt in range(4):
                pltpu.sync_copy(
                    in_hbm.at[pl.ds(dt * 8, 8), pl.ds(NGF * GS, NTAIL)],
                    tb.at[dt])

            @pl.loop(0, NTAIL)
            def _(r):
                rr = jnp.full((16,), r, jnp.int32)
                for dh in range(2):
                    v = plsc.load_gather(
                        tb, [lane // 8 + dh * 2, lane % 8, rr])
                    outb[0][pl.ds(r * D + dh * 16, 16)] = v
            pltpu.sync_copy(outb[0].at[pl.ds(0, NTAIL * 32)],
                            out_hbm.at[pl.ds(NGF * GS * 32, NTAIL * 32)])


def _linearize(his_t, item_t):
    mesh = plsc.VectorSubcoreMesh(core_axis_name="c", subcore_axis_name="s")
    fn = pl.kernel(
        _linearize_body,
        out_type=(jax.ShapeDtypeStruct((NI * D,), jnp.float32),
                  jax.ShapeDtypeStruct((NI * D,), jnp.float32)),
        mesh=mesh,
        compiler_params=pltpu.CompilerParams(
            needs_layout_passes=False, use_tc_tiling_on_sc=True),
        scratch_types=[
            pltpu.VMEM((D, GS + 1), jnp.float32),
            pltpu.VMEM((D, GS + 1), jnp.float32),
            pltpu.VMEM((GS * D,), jnp.float32),
            pltpu.VMEM((GS * D,), jnp.float32),
            pltpu.VMEM((4, 8, NTAIL), jnp.float32),
            pltpu.SemaphoreType.DMA,
            pltpu.SemaphoreType.DMA,
            pltpu.SemaphoreType.DMA,
            pltpu.SemaphoreType.DMA,
        ],
    )
    return fn(his_t, item_t)


def kernel(users, his_items, his_lens, pre_items, his_emb_table,
           item_emb_table, user_bias_table, item_bias_table):
    scale = jnp.power(his_lens, -0.5).astype(jnp.float32)
    his_flat = his_items.reshape(B * L).astype(jnp.int32)
    pre = pre_items.astype(jnp.int32)
    usr = users.astype(jnp.int32)
    ub = jnp.reshape(user_bias_table.T, (NI,))
    ib = jnp.reshape(item_bias_table.T, (NI,))
    his_lin, item_lin = _linearize(his_emb_table.T, item_emb_table.T)
    his_tab = his_lin.reshape(NI, D)
    item_tab = item_lin.reshape(NI, D)
    mesh = plsc.VectorSubcoreMesh(core_axis_name="c", subcore_axis_name="s")
    fn = pl.kernel(
        _fism_body,
        out_type=jax.ShapeDtypeStruct((B,), jnp.float32),
        mesh=mesh,
        compiler_params=pltpu.CompilerParams(
            needs_layout_passes=False, use_tc_tiling_on_sc=False),
        scratch_types=[
            pltpu.VMEM((HU * L,), jnp.int32),
            pltpu.VMEM((RG, D), jnp.float32),
            pltpu.VMEM((RG, D), jnp.float32),
            pltpu.VMEM((PB,), jnp.int32),
            pltpu.VMEM((PB,), jnp.int32),
            pltpu.VMEM((PB, D), jnp.float32),
            pltpu.VMEM((PB,), jnp.float32),
            pltpu.VMEM((PB,), jnp.float32),
            pltpu.VMEM((PB,), jnp.float32),
            pltpu.VMEM((PB,), jnp.float32),
            pltpu.VMEM((UB * 16,), jnp.float32),
            pltpu.SemaphoreType.DMA,
            pltpu.SemaphoreType.DMA,
            pltpu.SemaphoreType.DMA,
        ],
    )
    return fn(his_flat, pre, usr, scale, his_tab, item_tab, ub, ib)


# speedup vs baseline: 2.4027x; 1.0969x over previous
import jax
import jax.numpy as jnp
from jax import lax
from jax.experimental import pallas as pl
from jax.experimental.pallas import tpu as pltpu
from jax.experimental.pallas import tpu_sc as plsc

B = 16384
L = 200
D = 32
NC = 2
NS = 16
NW = NC * NS
PB = B // NW
G = 4
RG = G * L
UB = 16
HU = PB // 2
HB = HU // UB


def _fism_body(his_flat_hbm, pre_hbm, users_hbm, scale_hbm,
               his_tab_hbm, item_tab_hbm, ubias_hbm, ibias_hbm,
               out_hbm,
               idx_half, rows0, rows1, pre_v, users_v, item_rows, scale_v,
               ubias_v, ibias_v, out_v, prod_buf,
               semr0, semr1, sem2):
    wid = lax.axis_index("s") * NC + lax.axis_index("c")
    base = wid * PB
    rows = (rows0, rows1)
    sems = (semr0, semr1)

    pltpu.sync_copy(pre_hbm.at[pl.ds(base, PB)], pre_v)
    pltpu.sync_copy(users_hbm.at[pl.ds(base, PB)], users_v)
    pltpu.sync_copy(scale_hbm.at[pl.ds(base, PB)], scale_v)
    cp_items = pltpu.async_copy(item_tab_hbm.at[pre_v], item_rows, sem2)
    cp_ub = pltpu.async_copy(ubias_hbm.at[users_v], ubias_v, sem2)
    cp_ib = pltpu.async_copy(ibias_hbm.at[pre_v], ibias_v, sem2)

    lane16 = lax.iota(jnp.int32, 16) * 16

    def fire(sg, b):
        pltpu.async_copy(
            his_tab_hbm.at[idx_half.at[pl.ds(sg * RG, RG)]], rows[b], sems[b])

    def wait(b):
        pltpu.make_async_copy(
            his_tab_hbm.at[idx_half.at[pl.ds(0, RG)]], rows[b], sems[b]).wait()

    cp_items.wait()
    cp_ub.wait()
    cp_ib.wait()

    @pl.loop(0, 2)
    def _(h):
        ho = h * HU
        pltpu.sync_copy(his_flat_hbm.at[pl.ds((base + ho) * L, HU * L)],
                        idx_half)
        fire(0, 0)

        @pl.loop(0, HB)
        def _(blk):
            for s in range(4):
                b = s % 2
                sg = blk * 4 + s
                if s < 3:
                    fire(sg + 1, 1 - b)
                else:
                    @pl.when(blk < HB - 1)
                    def _():
                        fire(sg + 1, 1 - b)
                wait(b)
                rv = rows[b]

                @pl.loop(0, G)
                def _(u):
                    init = (jnp.zeros((16,), jnp.float32),) * 8

                    @pl.loop(0, L, step=8, unroll=5, init_carry=init)
                    def pool(j, accs):
                        accs = list(accs)
                        for k in range(8):
                            r = u * L + j + k
                            c = k % 4
                            accs[c] = accs[c] + rv[r, pl.ds(0, 16)]
                            accs[4 + c] = accs[4 + c] + rv[r, pl.ds(16, 16)]
                        return tuple(accs)

                    lo = (pool[0] + pool[1]) + (pool[2] + pool[3])
                    hi = (pool[4] + pool[5]) + (pool[6] + pool[7])
                    uu = ho + blk * UB + s * G + u
                    prod = (lo * item_rows[uu, pl.ds(0, 16)]
                            + hi * item_rows[uu, pl.ds(16, 16)])
                    prod_buf[pl.ds((s * G + u) * 16, 16)] = prod

            rating = jnp.zeros((16,), jnp.float32)
            for d in range(16):
                rating = rating + plsc.load_gather(prod_buf, [lane16 + d])
            b0 = ho + blk * UB
            rating = (rating * scale_v[pl.ds(b0, UB)]
                      + ubias_v[pl.ds(b0, UB)] + ibias_v[pl.ds(b0, UB)])
            out_v[pl.ds(b0, UB)] = rating

    pltpu.sync_copy(out_v, out_hbm.at[pl.ds(base, PB)])


NI = 1000000
GS = 512
NGF = NI // GS
NTAIL = NI - NGF * GS
NLI = (NGF // NW + 1 + 1) // 2


def _linearize_body(a_hbm, b_hbm, oa_hbm, ob_hbm,
                    st0, st1, ob0, ob1, tb,
                    isem0, isem1, osem0, osem1):
    wid = lax.axis_index("s") * NC + lax.axis_index("c")
    lane = lax.iota(jnp.int32, 16)
    lq4 = lane % 4
    sq4 = (lane // 4) * D + lane % 4
    stage = (st0, st1)
    outb = (ob0, ob1)
    isems = (isem0, isem1)
    osems = (osem0, osem1)

    for in_hbm, out_hbm in ((a_hbm, oa_hbm), (b_hbm, ob_hbm)):
        def fire_in(g, b):
            for dt in range(4):
                pltpu.async_copy(
                    in_hbm.at[pl.ds(dt * 8, 8), pl.ds(g * GS, GS)],
                    stage[b].at[pl.ds(dt * 8, 8), pl.ds(0, GS)], isems[b])

        def wait_in(b):
            for dt in range(4):
                pltpu.make_async_copy(
                    in_hbm.at[pl.ds(0, 8), pl.ds(0, GS)],
                    stage[b].at[pl.ds(dt * 8, 8), pl.ds(0, GS)],
                    isems[b]).wait()

        def wait_out(b):
            pltpu.make_async_copy(
                outb[b], out_hbm.at[pl.ds(0, GS * D)], osems[b]).wait()

        fire_in(wid, 0)

        @pl.loop(0, NLI)
        def _(li):
            for p in range(2):
                g = wid + (li * 2 + p) * NW

                @pl.when(g + NW < NGF)
                def _():
                    fire_in(g + NW, 1 - p)

                @pl.when(g < NGF)
                def _():
                    wait_in(p)

                    @pl.when(g >= 2 * NW)
                    def _():
                        wait_out(p)

                    @plsc.parallel_loop(0, GS // 4, unroll=4)
                    def _(rq):
                        r0 = rq * 4
                        for dq in range(8):
                            v = plsc.load_gather(
                                stage[p], [lq4 + dq * 4, lane // 4 + r0])
                            plsc.store_scatter(
                                outb[p],
                                [sq4 + (r0 * D + dq * 4)], v)

                    pltpu.async_copy(outb[p],
                                     out_hbm.at[pl.ds(g * (GS * D), GS * D)],
                                     osems[p])

        wait_out(0)
        wait_out(1)

        @pl.when(wid == 31)
        def _():
            for dt in range(4):
                pltpu.sync_copy(
                    in_hbm.at[pl.ds(dt * 8, 8), pl.ds(NGF * GS, NTAIL)],
                    tb.at[dt])

            @pl.loop(0, NTAIL)
            def _(r):
                rr = jnp.full((16,), r, jnp.int32)
                for dh in range(2):
                    v = plsc.load_gather(
                        tb, [lane // 8 + dh * 2, lane % 8, rr])
                    outb[0][pl.ds(r * D + dh * 16, 16)] = v
            pltpu.sync_copy(outb[0].at[pl.ds(0, NTAIL * 32)],
                            out_hbm.at[pl.ds(NGF * GS * 32, NTAIL * 32)])


def _linearize(his_t, item_t):
    mesh = plsc.VectorSubcoreMesh(core_axis_name="c", subcore_axis_name="s")
    fn = pl.kernel(
        _linearize_body,
        out_type=(jax.ShapeDtypeStruct((NI * D,), jnp.float32),
                  jax.ShapeDtypeStruct((NI * D,), jnp.float32)),
        mesh=mesh,
        compiler_params=pltpu.CompilerParams(
            needs_layout_passes=False, use_tc_tiling_on_sc=True),
        scratch_types=[
            pltpu.VMEM((D, GS + 1), jnp.float32),
            pltpu.VMEM((D, GS + 1), jnp.float32),
            pltpu.VMEM((GS * D,), jnp.float32),
            pltpu.VMEM((GS * D,), jnp.float32),
            pltpu.VMEM((4, 8, NTAIL), jnp.float32),
            pltpu.SemaphoreType.DMA,
            pltpu.SemaphoreType.DMA,
            pltpu.SemaphoreType.DMA,
            pltpu.SemaphoreType.DMA,
        ],
    )
    return fn(his_t, item_t)


def kernel(users, his_items, his_lens, pre_items, his_emb_table,
           item_emb_table, user_bias_table, item_bias_table):
    scale = jnp.power(his_lens, -0.5).astype(jnp.float32)
    his_flat = his_items.reshape(B * L).astype(jnp.int32)
    pre = pre_items.astype(jnp.int32)
    usr = users.astype(jnp.int32)
    ub = jnp.reshape(user_bias_table.T, (NI,))
    ib = jnp.reshape(item_bias_table.T, (NI,))
    his_lin, item_lin = _linearize(his_emb_table.T, item_emb_table.T)
    his_tab = his_lin.reshape(NI, D)
    item_tab = item_lin.reshape(NI, D)
    mesh = plsc.VectorSubcoreMesh(core_axis_name="c", subcore_axis_name="s")
    fn = pl.kernel(
        _fism_body,
        out_type=jax.ShapeDtypeStruct((B,), jnp.float32),
        mesh=mesh,
        compiler_params=pltpu.CompilerParams(
            needs_layout_passes=False, use_tc_tiling_on_sc=False),
        scratch_types=[
            pltpu.VMEM((HU * L,), jnp.int32),
            pltpu.VMEM((RG, D), jnp.float32),
            pltpu.VMEM((RG, D), jnp.float32),
            pltpu.VMEM((PB,), jnp.int32),
            pltpu.VMEM((PB,), jnp.int32),
            pltpu.VMEM((PB, D), jnp.float32),
            pltpu.VMEM((PB,), jnp.float32),
            pltpu.VMEM((PB,), jnp.float32),
            pltpu.VMEM((PB,), jnp.float32),
            pltpu.VMEM((PB,), jnp.float32),
            pltpu.VMEM((UB * 16,), jnp.float32),
            pltpu.SemaphoreType.DMA,
            pltpu.SemaphoreType.DMA,
            pltpu.SemaphoreType.DMA,
        ],
    )
    return fn(his_flat, pre, usr, scale, his_tab, item_tab, ub, ib)


# speedup vs baseline: 2.4654x; 1.0261x over previous
import jax
import jax.numpy as jnp
from jax import lax
from jax.experimental import pallas as pl
from jax.experimental.pallas import tpu as pltpu
from jax.experimental.pallas import tpu_sc as plsc

B = 16384
L = 200
D = 32
NC = 2
NS = 16
NW = NC * NS
PB = B // NW
G = 4
RG = G * L
UB = 16
HU = PB // 2
HB = HU // UB


def _fism_body(his_flat_hbm, pre_hbm, users_hbm, scale_hbm,
               his_tab_hbm, item_tab_hbm, ubias_hbm, ibias_hbm,
               out_hbm,
               idx_half, rows0, rows1, pre_v, users_v, item_rows, scale_v,
               ubias_v, ibias_v, out_v, prod_buf,
               semr0, semr1, sem2):
    wid = lax.axis_index("s") * NC + lax.axis_index("c")
    base = wid * PB
    rows = (rows0, rows1)
    sems = (semr0, semr1)

    pltpu.sync_copy(pre_hbm.at[pl.ds(base, PB)], pre_v)
    pltpu.sync_copy(users_hbm.at[pl.ds(base, PB)], users_v)
    pltpu.sync_copy(scale_hbm.at[pl.ds(base, PB)], scale_v)
    cp_items = pltpu.async_copy(item_tab_hbm.at[pre_v], item_rows, sem2)
    cp_ub = pltpu.async_copy(ubias_hbm.at[users_v], ubias_v, sem2)
    cp_ib = pltpu.async_copy(ibias_hbm.at[pre_v], ibias_v, sem2)

    lane16 = lax.iota(jnp.int32, 16) * 16

    def fire(sg, b):
        pltpu.async_copy(
            his_tab_hbm.at[idx_half.at[pl.ds(sg * RG, RG)]], rows[b], sems[b])

    def wait(b):
        pltpu.make_async_copy(
            his_tab_hbm.at[idx_half.at[pl.ds(0, RG)]], rows[b], sems[b]).wait()

    cp_items.wait()
    cp_ub.wait()
    cp_ib.wait()

    @pl.loop(0, 2)
    def _(h):
        ho = h * HU
        pltpu.sync_copy(his_flat_hbm.at[pl.ds((base + ho) * L, HU * L)],
                        idx_half)
        fire(0, 0)

        @pl.loop(0, HB)
        def _(blk):
            for s in range(4):
                b = s % 2
                sg = blk * 4 + s
                if s < 3:
                    fire(sg + 1, 1 - b)
                else:
                    @pl.when(blk < HB - 1)
                    def _():
                        fire(sg + 1, 1 - b)
                wait(b)
                rv = rows[b]

                @pl.loop(0, G)
                def _(u):
                    init = (jnp.zeros((16,), jnp.float32),) * 8

                    @pl.loop(0, L, step=8, unroll=5, init_carry=init)
                    def pool(j, accs):
                        accs = list(accs)
                        for k in range(8):
                            r = u * L + j + k
                            c = k % 4
                            accs[c] = accs[c] + rv[r, pl.ds(0, 16)]
                            accs[4 + c] = accs[4 + c] + rv[r, pl.ds(16, 16)]
                        return tuple(accs)

                    lo = (pool[0] + pool[1]) + (pool[2] + pool[3])
                    hi = (pool[4] + pool[5]) + (pool[6] + pool[7])
                    uu = ho + blk * UB + s * G + u
                    prod = (lo * item_rows[uu, pl.ds(0, 16)]
                            + hi * item_rows[uu, pl.ds(16, 16)])
                    prod_buf[pl.ds((s * G + u) * 16, 16)] = prod

            rating = jnp.zeros((16,), jnp.float32)
            for d in range(16):
                rating = rating + plsc.load_gather(prod_buf, [lane16 + d])
            b0 = ho + blk * UB
            rating = (rating * scale_v[pl.ds(b0, UB)]
                      + ubias_v[pl.ds(b0, UB)] + ibias_v[pl.ds(b0, UB)])
            out_v[pl.ds(b0, UB)] = rating

    pltpu.sync_copy(out_v, out_hbm.at[pl.ds(base, PB)])


NI = 1000000
GS = 768
NGF = NI // GS
NTAIL = NI - NGF * GS
NLI = (NGF // NW + 1 + 1) // 2


def _linearize_body(a_hbm, b_hbm, oa_hbm, ob_hbm,
                    st0, st1, ob0, ob1, tb,
                    isem0, isem1, osem0, osem1):
    wid = lax.axis_index("s") * NC + lax.axis_index("c")
    lane = lax.iota(jnp.int32, 16)
    lq4 = lane % 4
    sq4 = (lane // 4) * D + lane % 4
    stage = (st0, st1)
    outb = (ob0, ob1)
    isems = (isem0, isem1)
    osems = (osem0, osem1)

    for in_hbm, out_hbm in ((a_hbm, oa_hbm), (b_hbm, ob_hbm)):
        def fire_in(g, b):
            for dt in range(4):
                pltpu.async_copy(
                    in_hbm.at[pl.ds(dt * 8, 8), pl.ds(g * GS, GS)],
                    stage[b].at[pl.ds(dt * 8, 8), pl.ds(0, GS)], isems[b])

        def wait_in(b):
            for dt in range(4):
                pltpu.make_async_copy(
                    in_hbm.at[pl.ds(0, 8), pl.ds(0, GS)],
                    stage[b].at[pl.ds(dt * 8, 8), pl.ds(0, GS)],
                    isems[b]).wait()

        def wait_out(b):
            pltpu.make_async_copy(
                outb[b], out_hbm.at[pl.ds(0, GS * D)], osems[b]).wait()

        fire_in(wid, 0)

        @pl.loop(0, NLI)
        def _(li):
            for p in range(2):
                g = wid + (li * 2 + p) * NW

                @pl.when(g + NW < NGF)
                def _():
                    fire_in(g + NW, 1 - p)

                @pl.when(g < NGF)
                def _():
                    wait_in(p)

                    @pl.when(g >= 2 * NW)
                    def _():
                        wait_out(p)

                    @plsc.parallel_loop(0, GS // 4, unroll=4)
                    def _(rq):
                        r0 = rq * 4
                        for dq in range(8):
                            v = plsc.load_gather(
                                stage[p], [lq4 + dq * 4, lane // 4 + r0])
                            plsc.store_scatter(
                                outb[p],
                                [sq4 + (r0 * D + dq * 4)], v)

                    pltpu.async_copy(outb[p],
                                     out_hbm.at[pl.ds(g * (GS * D), GS * D)],
                                     osems[p])

        wait_out(0)
        wait_out(1)

        @pl.when(wid == 31)
        def _():
            for dt in range(4):
                pltpu.sync_copy(
                    in_hbm.at[pl.ds(dt * 8, 8), pl.ds(NGF * GS, NTAIL)],
                    tb.at[dt])

            @pl.loop(0, NTAIL)
            def _(r):
                rr = jnp.full((16,), r, jnp.int32)
                for dh in range(2):
                    v = plsc.load_gather(
                        tb, [lane // 8 + dh * 2, lane % 8, rr])
                    outb[0][pl.ds(r * D + dh * 16, 16)] = v
            pltpu.sync_copy(outb[0].at[pl.ds(0, NTAIL * 32)],
                            out_hbm.at[pl.ds(NGF * GS * 32, NTAIL * 32)])


def _linearize(his_t, item_t):
    mesh = plsc.VectorSubcoreMesh(core_axis_name="c", subcore_axis_name="s")
    fn = pl.kernel(
        _linearize_body,
        out_type=(jax.ShapeDtypeStruct((NI * D,), jnp.float32),
                  jax.ShapeDtypeStruct((NI * D,), jnp.float32)),
        mesh=mesh,
        compiler_params=pltpu.CompilerParams(
            needs_layout_passes=False, use_tc_tiling_on_sc=True),
        scratch_types=[
            pltpu.VMEM((D, GS + 1), jnp.float32),
            pltpu.VMEM((D, GS + 1), jnp.float32),
            pltpu.VMEM((GS * D,), jnp.float32),
            pltpu.VMEM((GS * D,), jnp.float32),
            pltpu.VMEM((4, 8, NTAIL), jnp.float32),
            pltpu.SemaphoreType.DMA,
            pltpu.SemaphoreType.DMA,
            pltpu.SemaphoreType.DMA,
            pltpu.SemaphoreType.DMA,
        ],
    )
    return fn(his_t, item_t)


def kernel(users, his_items, his_lens, pre_items, his_emb_table,
           item_emb_table, user_bias_table, item_bias_table):
    scale = jnp.power(his_lens, -0.5).astype(jnp.float32)
    his_flat = his_items.reshape(B * L).astype(jnp.int32)
    pre = pre_items.astype(jnp.int32)
    usr = users.astype(jnp.int32)
    ub = jnp.reshape(user_bias_table.T, (NI,))
    ib = jnp.reshape(item_bias_table.T, (NI,))
    his_lin, item_lin = _linearize(his_emb_table.T, item_emb_table.T)
    his_tab = his_lin.reshape(NI, D)
    item_tab = item_lin.reshape(NI, D)
    mesh = plsc.VectorSubcoreMesh(core_axis_name="c", subcore_axis_name="s")
    fn = pl.kernel(
        _fism_body,
        out_type=jax.ShapeDtypeStruct((B,), jnp.float32),
        mesh=mesh,
        compiler_params=pltpu.CompilerParams(
            needs_layout_passes=False, use_tc_tiling_on_sc=False),
        scratch_types=[
            pltpu.VMEM((HU * L,), jnp.int32),
            pltpu.VMEM((RG, D), jnp.float32),
            pltpu.VMEM((RG, D), jnp.float32),
            pltpu.VMEM((PB,), jnp.int32),
            pltpu.VMEM((PB,), jnp.int32),
            pltpu.VMEM((PB, D), jnp.float32),
            pltpu.VMEM((PB,), jnp.float32),
            pltpu.VMEM((PB,), jnp.float32),
            pltpu.VMEM((PB,), jnp.float32),
            pltpu.VMEM((PB,), jnp.float32),
            pltpu.VMEM((UB * 16,), jnp.float32),
            pltpu.SemaphoreType.DMA,
            pltpu.SemaphoreType.DMA,
            pltpu.SemaphoreType.DMA,
        ],
    )
    return fn(his_flat, pre, usr, scale, his_tab, item_tab, ub, ib)
